# baseline (speedup 1.0000x reference)
import jax
import jax.numpy as jnp
from jax import lax
from jax.experimental import pallas as pl
from jax.experimental.pallas import tpu as pltpu
from jax.experimental.pallas import tpu_sc as plsc

NUM_ITEMS = 1000000
LOOKBACK = 200
EMB_SIZE = 64
BATCH = 4096
SEQ_LEN = 200

NC = 2
NS = 16
NW = NC * NS
LANES = 16
VPR = EMB_SIZE // LANES

TOTAL_ROWS = BATCH * SEQ_LEN
ROWS_W = TOTAL_ROWS // NW
SEQS_W = ROWS_W // SEQ_LEN
NBUF = 4
AHEAD = 2

BLK = 128
NBLK = NUM_ITEMS // BLK
TAIL = NUM_ITEMS - NBLK * BLK
PACKED_ROWS = NUM_ITEMS // 2


def _fmt_body(tableT_hbm, tail_hbm, out_hbm, panel_v, blk_v, tail_v,
              gsem, osem):
    wid = lax.axis_index("s") * NC + lax.axis_index("c")

    iota = lax.iota(jnp.int32, 16)
    zero16 = iota * 0
    d_idx = [iota + (j % VPR) * LANES for j in range(2 * VPR)]

    def issue_read(blk, q):
        base = pl.multiple_of(blk * BLK, BLK)
        for dhi in range(EMB_SIZE // 8):
            pltpu.async_copy(
                tableT_hbm.at[pl.ds(dhi * 8, 8), pl.ds(base, BLK)],
                panel_v.at[q, pl.ds(dhi * 8, 8), :],
                gsem.at[q],
            )

    def wait_read(q):
        for dhi in range(EMB_SIZE // 8):
            pltpu.make_async_copy(
                tableT_hbm.at[pl.ds(0, 8), pl.ds(0, BLK)],
                panel_v.at[q, pl.ds(0, 8), :],
                gsem.at[q],
            ).wait()

    @pl.when(wid == 0)
    def _tail():
        pltpu.sync_copy(tail_hbm, tail_v)
        pltpu.sync_copy(tail_v, out_hbm.at[pl.ds(PACKED_ROWS - TAIL // 2,
                                                 TAIL // 2), :])

    NB_W = (NBLK + NW - 1) // NW

    issue_read(wid, 0)

    @pl.loop(0, NB_W + 1, step=2)
    def _group(go):
        for b in range(2):
            i = go + b
            blk = i * NW + wid

            @pl.when(blk < NBLK)
            def _do():
                wait_read(b)

                @pl.when((i + 1) * NW + wid < NBLK)
                def _next():
                    @pl.when(i >= 1)
                    def _drain():
                        pltpu.make_async_copy(
                            blk_v.at[1 - b],
                            out_hbm.at[pl.ds(0, BLK // 2), :],
                            osem.at[1 - b],
                        ).wait()

                    issue_read((i + 1) * NW + wid, 1 - b)

                @pl.loop(0, BLK // 2, unroll=8)
                def _row(kl):
                    r_even = zero16 + 2 * kl
                    r_odd = r_even + 1
                    for j in range(2 * VPR):
                        vals = plsc.load_gather(
                            panel_v.at[b],
                            [d_idx[j], r_odd if j >= VPR else r_even])
                        blk_v[b, kl, pl.ds(j * LANES, LANES)] = vals

                pltpu.async_copy(
                    blk_v.at[b],
                    out_hbm.at[pl.ds(blk * (BLK // 2), BLK // 2), :],
                    osem.at[b],
                )

    for b in range(2):
        pltpu.make_async_copy(
            blk_v.at[b],
            out_hbm.at[pl.ds(0, BLK // 2), :],
            osem.at[b],
        ).wait()


def _sc_body(idx_hbm, item_hbm, pos_hbm, out_hbm, idx_v, pos_v, rows_v,
             gsem, osem):
    wid = lax.axis_index("s") * NC + lax.axis_index("c")
    base = wid * ROWS_W
    seq0 = wid * SEQS_W

    pltpu.sync_copy(idx_hbm.at[pl.ds(base, ROWS_W)], idx_v)
    pltpu.sync_copy(pos_hbm, pos_v)

    def issue_gather(g, q):
        pltpu.async_copy(
            item_hbm.at[idx_v.at[pl.ds(g * SEQ_LEN, SEQ_LEN)]],
            rows_v.at[q],
            gsem.at[q],
        )

    for b in range(AHEAD):
        issue_gather(b, b)

    @pl.loop(0, SEQS_W, step=NBUF)
    def _group(go):
        for b in range(NBUF):
            g = go + b
            q = (b + AHEAD) % NBUF

            @pl.when(g + AHEAD < SEQS_W)
            def _issue():
                @pl.when(g >= NBUF - AHEAD)
                def _drain():
                    pltpu.make_async_copy(
                        rows_v.at[q],
                        out_hbm.at[0],
                        osem.at[q],
                    ).wait()

                issue_gather(g + AHEAD, q)

            pltpu.make_async_copy(
                item_hbm.at[idx_v.at[pl.ds(0, SEQ_LEN)]],
                rows_v.at[b],
                gsem.at[b],
            ).wait()

            @pl.loop(0, SEQ_LEN, unroll=8)
            def _row(l):
                for j in range(VPR):
                    sl = pl.ds(j * LANES, LANES)
                    rows_v[b, l, sl] = rows_v[b, l, sl] + pos_v[l, sl]

            pltpu.async_copy(
                rows_v.at[b],
                out_hbm.at[seq0 + g],
                osem.at[b],
            )

    for b in range(NBUF):
        pltpu.make_async_copy(
            rows_v.at[b],
            out_hbm.at[0],
            osem.at[b],
        ).wait()


@jax.jit
def _run(input_seqs, item_table, pos_table):
    mesh = plsc.VectorSubcoreMesh(core_axis_name="c", subcore_axis_name="s")

    fmt = pl.kernel(
        _fmt_body,
        out_type=jax.ShapeDtypeStruct((PACKED_ROWS, 128), jnp.float32),
        compiler_params=pltpu.CompilerParams(use_tc_tiling_on_sc=True, needs_layout_passes=False),
        mesh=mesh,
        scratch_types=[
            pltpu.VMEM((2, EMB_SIZE, BLK), jnp.float32),
            pltpu.VMEM((2, BLK // 2, 128), jnp.float32),
            pltpu.VMEM((TAIL // 2, 128), jnp.float32),
            pltpu.SemaphoreType.DMA((2,)),
            pltpu.SemaphoreType.DMA((2,)),
        ],
    )
    tail_packed = item_table[NBLK * BLK:, :].reshape(TAIL // 2, 128)
    packed = fmt(item_table.T, tail_packed)
    item_lin = packed.reshape(NUM_ITEMS, EMB_SIZE)

    idx_flat = input_seqs.reshape(-1).astype(jnp.int32)
    k = pl.kernel(
        _sc_body,
        out_type=jax.ShapeDtypeStruct((BATCH, SEQ_LEN, EMB_SIZE), jnp.float32),
        mesh=mesh,
        scratch_types=[
            pltpu.VMEM((ROWS_W,), jnp.int32),
            pltpu.VMEM((LOOKBACK, EMB_SIZE), jnp.float32),
            pltpu.VMEM((NBUF, SEQ_LEN, EMB_SIZE), jnp.float32),
            pltpu.SemaphoreType.DMA((NBUF,)),
            pltpu.SemaphoreType.DMA((NBUF,)),
        ],
        compiler_params=pltpu.CompilerParams(use_tc_tiling_on_sc=False),
    )
    return k(idx_flat, item_lin, pos_table)


def kernel(input_seqs, item_table, pos_table):
    return _run(input_seqs, item_table, pos_table)


# speedup vs baseline: 1.4225x; 1.4225x over previous
import jax
import jax.numpy as jnp
from jax import lax
from jax.experimental import pallas as pl
from jax.experimental.pallas import tpu as pltpu
from jax.experimental.pallas import tpu_sc as plsc

NUM_ITEMS = 1000000
LOOKBACK = 200
EMB_SIZE = 64
BATCH = 4096
SEQ_LEN = 200

NC = 2
NS = 16
NW = NC * NS
LANES = 16
VPR = EMB_SIZE // LANES

TOTAL_ROWS = BATCH * SEQ_LEN
ROWS_W = TOTAL_ROWS // NW
SEQS_W = ROWS_W // SEQ_LEN
NBUF = 4
AHEAD = 2

BLK = 128
NBLK = NUM_ITEMS // BLK
TAIL = NUM_ITEMS - NBLK * BLK
PACKED_ROWS = NUM_ITEMS // 2


def _fmt_body(tableT_hbm, tail_hbm, out_hbm, panel_v, blk_v, tail_v,
              gsem, osem):
    wid = lax.axis_index("s") * NC + lax.axis_index("c")

    iota = lax.iota(jnp.int32, 16)
    zero16 = iota * 0
    d_idx = [iota + (j % VPR) * LANES for j in range(2 * VPR)]

    def issue_read(blk, q):
        base = pl.multiple_of(blk * BLK, BLK)
        for dhi in range(EMB_SIZE // 8):
            pltpu.async_copy(
                tableT_hbm.at[pl.ds(dhi * 8, 8), pl.ds(base, BLK)],
                panel_v.at[q, pl.ds(dhi * 8, 8), :],
                gsem.at[q],
            )

    def wait_read(q):
        for dhi in range(EMB_SIZE // 8):
            pltpu.make_async_copy(
                tableT_hbm.at[pl.ds(0, 8), pl.ds(0, BLK)],
                panel_v.at[q, pl.ds(0, 8), :],
                gsem.at[q],
            ).wait()

    @pl.when(wid == 0)
    def _tail():
        pltpu.sync_copy(tail_hbm, tail_v)
        pltpu.sync_copy(tail_v, out_hbm.at[pl.ds(PACKED_ROWS - TAIL // 2,
                                                 TAIL // 2), :])

    NB_W = (NBLK + NW - 1) // NW

    issue_read(wid, 0)

    @pl.loop(0, NB_W + 1, step=2)
    def _group(go):
        for b in range(2):
            i = go + b
            blk = i * NW + wid

            @pl.when(blk < NBLK)
            def _do():
                wait_read(b)

                @pl.when((i + 1) * NW + wid < NBLK)
                def _next():
                    @pl.when(i >= 1)
                    def _drain():
                        pltpu.make_async_copy(
                            blk_v.at[1 - b],
                            out_hbm.at[pl.ds(0, BLK // 2), :],
                            osem.at[1 - b],
                        ).wait()

                    issue_read((i + 1) * NW + wid, 1 - b)

                @plsc.parallel_loop(0, BLK // 2, unroll=8)
                def _row(kl):
                    r_even = zero16 + 2 * kl
                    r_odd = r_even + 1
                    for j in range(2 * VPR):
                        vals = plsc.load_gather(
                            panel_v.at[b],
                            [d_idx[j], r_odd if j >= VPR else r_even])
                        blk_v[b, kl, pl.ds(j * LANES, LANES)] = vals

                pltpu.async_copy(
                    blk_v.at[b],
                    out_hbm.at[pl.ds(blk * (BLK // 2), BLK // 2), :],
                    osem.at[b],
                )

    for b in range(2):
        pltpu.make_async_copy(
            blk_v.at[b],
            out_hbm.at[pl.ds(0, BLK // 2), :],
            osem.at[b],
        ).wait()


def _sc_body(idx_hbm, item_hbm, pos_hbm, out_hbm, idx_v, pos_v, rows_v,
             gsem, osem):
    wid = lax.axis_index("s") * NC + lax.axis_index("c")
    base = wid * ROWS_W
    seq0 = wid * SEQS_W

    pltpu.sync_copy(idx_hbm.at[pl.ds(base, ROWS_W)], idx_v)
    pltpu.sync_copy(pos_hbm, pos_v)

    def issue_gather(g, q):
        pltpu.async_copy(
            item_hbm.at[idx_v.at[pl.ds(g * SEQ_LEN, SEQ_LEN)]],
            rows_v.at[q],
            gsem.at[q],
        )

    for b in range(AHEAD):
        issue_gather(b, b)

    @pl.loop(0, SEQS_W, step=NBUF)
    def _group(go):
        for b in range(NBUF):
            g = go + b
            q = (b + AHEAD) % NBUF

            @pl.when(g + AHEAD < SEQS_W)
            def _issue():
                @pl.when(g >= NBUF - AHEAD)
                def _drain():
                    pltpu.make_async_copy(
                        rows_v.at[q],
                        out_hbm.at[0],
                        osem.at[q],
                    ).wait()

                issue_gather(g + AHEAD, q)

            pltpu.make_async_copy(
                item_hbm.at[idx_v.at[pl.ds(0, SEQ_LEN)]],
                rows_v.at[b],
                gsem.at[b],
            ).wait()

            @pl.loop(0, SEQ_LEN, unroll=8)
            def _row(l):
                for j in range(VPR):
                    sl = pl.ds(j * LANES, LANES)
                    rows_v[b, l, sl] = rows_v[b, l, sl] + pos_v[l, sl]

            pltpu.async_copy(
                rows_v.at[b],
                out_hbm.at[seq0 + g],
                osem.at[b],
            )

    for b in range(NBUF):
        pltpu.make_async_copy(
            rows_v.at[b],
            out_hbm.at[0],
            osem.at[b],
        ).wait()


@jax.jit
def _run(input_seqs, item_table, pos_table):
    mesh = plsc.VectorSubcoreMesh(core_axis_name="c", subcore_axis_name="s")

    fmt = pl.kernel(
        _fmt_body,
        out_type=jax.ShapeDtypeStruct((PACKED_ROWS, 128), jnp.float32),
        compiler_params=pltpu.CompilerParams(use_tc_tiling_on_sc=True, needs_layout_passes=False),
        mesh=mesh,
        scratch_types=[
            pltpu.VMEM((2, EMB_SIZE, BLK), jnp.float32),
            pltpu.VMEM((2, BLK // 2, 128), jnp.float32),
            pltpu.VMEM((TAIL // 2, 128), jnp.float32),
            pltpu.SemaphoreType.DMA((2,)),
            pltpu.SemaphoreType.DMA((2,)),
        ],
    )
    tail_packed = item_table[NBLK * BLK:, :].reshape(TAIL // 2, 128)
    packed = fmt(item_table.T, tail_packed)
    item_lin = packed.reshape(NUM_ITEMS, EMB_SIZE)

    idx_flat = input_seqs.reshape(-1).astype(jnp.int32)
    k = pl.kernel(
        _sc_body,
        out_type=jax.ShapeDtypeStruct((BATCH, SEQ_LEN, EMB_SIZE), jnp.float32),
        mesh=mesh,
        scratch_types=[
            pltpu.VMEM((ROWS_W,), jnp.int32),
            pltpu.VMEM((LOOKBACK, EMB_SIZE), jnp.float32),
            pltpu.VMEM((NBUF, SEQ_LEN, EMB_SIZE), jnp.float32),
            pltpu.SemaphoreType.DMA((NBUF,)),
            pltpu.SemaphoreType.DMA((NBUF,)),
        ],
        compiler_params=pltpu.CompilerParams(use_tc_tiling_on_sc=False),
    )
    return k(idx_flat, item_lin, pos_table)


def kernel(input_seqs, item_table, pos_table):
    return _run(input_seqs, item_table, pos_table)


# speedup vs baseline: 1.5657x; 1.1007x over previous
import jax
import jax.numpy as jnp
from jax import lax
from jax.experimental import pallas as pl
from jax.experimental.pallas import tpu as pltpu
from jax.experimental.pallas import tpu_sc as plsc

NUM_ITEMS = 1000000
LOOKBACK = 200
EMB_SIZE = 64
BATCH = 4096
SEQ_LEN = 200

NC = 2
NS = 16
NW = NC * NS
LANES = 16
VPR = EMB_SIZE // LANES

TOTAL_ROWS = BATCH * SEQ_LEN
ROWS_W = TOTAL_ROWS // NW
SEQS_W = ROWS_W // SEQ_LEN
NBUF = 4
AHEAD = 2

BLK = 128
NBLK = NUM_ITEMS // BLK
TAIL = NUM_ITEMS - NBLK * BLK
PACKED_ROWS = NUM_ITEMS // 2


def _fmt_body(tableT_hbm, tail_hbm, out_hbm, panel_v, blk_v, tail_v,
              gsem, osem):
    wid = lax.axis_index("s") * NC + lax.axis_index("c")

    iota = lax.iota(jnp.int32, 16)
    zero16 = iota * 0
    d_idx = [iota + (j % VPR) * LANES for j in range(2 * VPR)]

    def issue_read(blk, q):
        base = pl.multiple_of(blk * BLK, BLK)
        for dhi in range(EMB_SIZE // 8):
            pltpu.async_copy(
                tableT_hbm.at[pl.ds(dhi * 8, 8), pl.ds(base, BLK)],
                panel_v.at[q, pl.ds(dhi * 8, 8), :],
                gsem.at[q],
            )

    def wait_read(q):
        for dhi in range(EMB_SIZE // 8):
            pltpu.make_async_copy(
                tableT_hbm.at[pl.ds(0, 8), pl.ds(0, BLK)],
                panel_v.at[q, pl.ds(0, 8), :],
                gsem.at[q],
            ).wait()

    @pl.when(wid == 0)
    def _tail():
        pltpu.sync_copy(tail_hbm, tail_v)
        pltpu.sync_copy(tail_v, out_hbm.at[pl.ds(PACKED_ROWS - TAIL // 2,
                                                 TAIL // 2), :])

    NB_W = (NBLK + NW - 1) // NW

    issue_read(wid, 0)

    @pl.loop(0, NB_W + 1, step=2)
    def _group(go):
        for b in range(2):
            i = go + b
            blk = i * NW + wid

            @pl.when(blk < NBLK)
            def _do():
                wait_read(b)

                @pl.when((i + 1) * NW + wid < NBLK)
                def _next():
                    @pl.when(i >= 1)
                    def _drain():
                        pltpu.make_async_copy(
                            blk_v.at[1 - b],
                            out_hbm.at[pl.ds(0, BLK // 2), :],
                            osem.at[1 - b],
                        ).wait()

                    issue_read((i + 1) * NW + wid, 1 - b)

                @plsc.parallel_loop(0, BLK // 2, unroll=8)
                def _row(kl):
                    r_even = zero16 + 2 * kl
                    r_odd = r_even + 1
                    for j in range(2 * VPR):
                        vals = plsc.load_gather(
                            panel_v.at[b],
                            [d_idx[j], r_odd if j >= VPR else r_even])
                        blk_v[b, kl, pl.ds(j * LANES, LANES)] = vals

                pltpu.async_copy(
                    blk_v.at[b],
                    out_hbm.at[pl.ds(blk * (BLK // 2), BLK // 2), :],
                    osem.at[b],
                )

    for b in range(2):
        pltpu.make_async_copy(
            blk_v.at[b],
            out_hbm.at[pl.ds(0, BLK // 2), :],
            osem.at[b],
        ).wait()


def _sc_body(idx_hbm, item_hbm, pos_hbm, out_hbm, idx_v, pos_v, rows_v,
             gsem, osem):
    wid = lax.axis_index("s") * NC + lax.axis_index("c")
    base = wid * ROWS_W
    seq0 = wid * SEQS_W

    pltpu.sync_copy(idx_hbm.at[pl.ds(base, ROWS_W)], idx_v)
    pltpu.sync_copy(pos_hbm, pos_v)

    def issue_gather(g, q):
        pltpu.async_copy(
            item_hbm.at[idx_v.at[pl.ds(g * SEQ_LEN, SEQ_LEN)]],
            rows_v.at[q],
            gsem.at[q],
        )

    for b in range(AHEAD):
        issue_gather(b, b)

    @pl.loop(0, SEQS_W, step=NBUF)
    def _group(go):
        for b in range(NBUF):
            g = go + b
            q = (b + AHEAD) % NBUF

            @pl.when(g + AHEAD < SEQS_W)
            def _issue():
                @pl.when(g >= NBUF - AHEAD)
                def _drain():
                    pltpu.make_async_copy(
                        rows_v.at[q],
                        out_hbm.at[0],
                        osem.at[q],
                    ).wait()

                issue_gather(g + AHEAD, q)

            pltpu.make_async_copy(
                item_hbm.at[idx_v.at[pl.ds(0, SEQ_LEN)]],
                rows_v.at[b],
                gsem.at[b],
            ).wait()

            @plsc.parallel_loop(0, SEQ_LEN, unroll=8)
            def _row(l):
                for j in range(VPR):
                    sl = pl.ds(j * LANES, LANES)
                    rows_v[b, l, sl] = rows_v[b, l, sl] + pos_v[l, sl]

            pltpu.async_copy(
                rows_v.at[b],
                out_hbm.at[seq0 + g],
                osem.at[b],
            )

    for b in range(NBUF):
        pltpu.make_async_copy(
            rows_v.at[b],
            out_hbm.at[0],
            osem.at[b],
        ).wait()


@jax.jit
def _run(input_seqs, item_table, pos_table):
    mesh = plsc.VectorSubcoreMesh(core_axis_name="c", subcore_axis_name="s")

    fmt = pl.kernel(
        _fmt_body,
        out_type=jax.ShapeDtypeStruct((PACKED_ROWS, 128), jnp.float32),
        compiler_params=pltpu.CompilerParams(use_tc_tiling_on_sc=True, needs_layout_passes=False),
        mesh=mesh,
        scratch_types=[
            pltpu.VMEM((2, EMB_SIZE, BLK), jnp.float32),
            pltpu.VMEM((2, BLK // 2, 128), jnp.float32),
            pltpu.VMEM((TAIL // 2, 128), jnp.float32),
            pltpu.SemaphoreType.DMA((2,)),
            pltpu.SemaphoreType.DMA((2,)),
        ],
    )
    tail_packed = item_table[NBLK * BLK:, :].reshape(TAIL // 2, 128)
    packed = fmt(item_table.T, tail_packed)
    item_lin = packed.reshape(NUM_ITEMS, EMB_SIZE)

    idx_flat = input_seqs.reshape(-1).astype(jnp.int32)
    k = pl.kernel(
        _sc_body,
        out_type=jax.ShapeDtypeStruct((BATCH, SEQ_LEN, EMB_SIZE), jnp.float32),
        mesh=mesh,
        scratch_types=[
            pltpu.VMEM((ROWS_W,), jnp.int32),
            pltpu.VMEM((LOOKBACK, EMB_SIZE), jnp.float32),
            pltpu.VMEM((NBUF, SEQ_LEN, EMB_SIZE), jnp.float32),
            pltpu.SemaphoreType.DMA((NBUF,)),
            pltpu.SemaphoreType.DMA((NBUF,)),
        ],
        compiler_params=pltpu.CompilerParams(use_tc_tiling_on_sc=False),
    )
    return k(idx_flat, item_lin, pos_table)


def kernel(input_seqs, item_table, pos_table):
    return _run(input_seqs, item_table, pos_table)


# speedup vs baseline: 1.9201x; 1.2264x over previous
import jax
import jax.numpy as jnp
from jax import lax
from jax.experimental import pallas as pl
from jax.experimental.pallas import tpu as pltpu
from jax.experimental.pallas import tpu_sc as plsc

NUM_ITEMS = 1000000
LOOKBACK = 200
EMB_SIZE = 64
BATCH = 4096
SEQ_LEN = 200

NC = 2
NS = 16
NW = NC * NS
LANES = 16
VPR = EMB_SIZE // LANES

TOTAL_ROWS = BATCH * SEQ_LEN
ROWS_W = TOTAL_ROWS // NW
SEQS_W = ROWS_W // SEQ_LEN
NBUF = 4
AHEAD = 2


def _sc_body(idx_hbm, item_hbm, pos_hbm, out_hbm, idx_v, pos_v, rows_v,
             gsem, osem):
    wid = lax.axis_index("s") * NC + lax.axis_index("c")
    base = wid * ROWS_W
    seq0 = wid * SEQS_W

    pltpu.sync_copy(idx_hbm.at[pl.ds(base, ROWS_W)], idx_v)
    pltpu.sync_copy(pos_hbm, pos_v)

    def issue_gather(g, q):
        pltpu.async_copy(
            item_hbm.at[idx_v.at[pl.ds(g * SEQ_LEN, SEQ_LEN)]],
            rows_v.at[q],
            gsem.at[q],
        )

    for b in range(AHEAD):
        issue_gather(b, b)

    @pl.loop(0, SEQS_W, step=NBUF)
    def _group(go):
        for b in range(NBUF):
            g = go + b
            q = (b + AHEAD) % NBUF

            @pl.when(g + AHEAD < SEQS_W)
            def _issue():
                @pl.when(g >= NBUF - AHEAD)
                def _drain():
                    pltpu.make_async_copy(
                        rows_v.at[q],
                        out_hbm.at[0],
                        osem.at[q],
                    ).wait()

                issue_gather(g + AHEAD, q)

            pltpu.make_async_copy(
                item_hbm.at[idx_v.at[pl.ds(0, SEQ_LEN)]],
                rows_v.at[b],
                gsem.at[b],
            ).wait()

            @plsc.parallel_loop(0, SEQ_LEN, unroll=8)
            def _row(l):
                for j in range(VPR):
                    sl = pl.ds(j * LANES, LANES)
                    rows_v[b, l, sl] = rows_v[b, l, sl] + pos_v[l, sl]

            pltpu.async_copy(
                rows_v.at[b],
                out_hbm.at[seq0 + g],
                osem.at[b],
            )

    for b in range(NBUF):
        pltpu.make_async_copy(
            rows_v.at[b],
            out_hbm.at[0],
            osem.at[b],
        ).wait()


@jax.jit
def _run(input_seqs, item_table, pos_table):
    mesh = plsc.VectorSubcoreMesh(core_axis_name="c", subcore_axis_name="s")
    idx_flat = input_seqs.reshape(-1).astype(jnp.int32)
    k = pl.kernel(
        _sc_body,
        out_type=jax.ShapeDtypeStruct((BATCH, SEQ_LEN, EMB_SIZE), jnp.float32),
        mesh=mesh,
        scratch_types=[
            pltpu.VMEM((ROWS_W,), jnp.int32),
            pltpu.VMEM((LOOKBACK, EMB_SIZE), jnp.float32),
            pltpu.VMEM((NBUF, SEQ_LEN, EMB_SIZE), jnp.float32),
            pltpu.SemaphoreType.DMA((NBUF,)),
            pltpu.SemaphoreType.DMA((NBUF,)),
        ],
        compiler_params=pltpu.CompilerParams(use_tc_tiling_on_sc=False),
    )
    return k(idx_flat, item_table, pos_table)


def kernel(input_seqs, item_table, pos_table):
    return _run(input_seqs, item_table, pos_table)
